# baseline (speedup 1.0000x reference)
import functools

import jax
import jax.numpy as jnp
from jax import lax
from jax.experimental import pallas as pl
from jax.experimental.pallas import tpu as pltpu
from jax.experimental.pallas import tpu_sc as plsc

M_ROWS = 16384
OUT_F = 64
HALF_F = 32
K = 128
NSUB = 16
ROWS_PER_TILE = M_ROWS // NSUB
OVERRUN = 8 * K
WPAD = 16384


@functools.lru_cache(maxsize=None)
def _spmm_kernel(n_rows_w, per_tile, nc):
    mesh = plsc.VectorSubcoreMesh(core_axis_name="c", subcore_axis_name="s")

    @functools.partial(
        pl.kernel,
        mesh=mesh,
        compiler_params=pltpu.CompilerParams(use_tc_tiling_on_sc=False),
        out_type=jax.ShapeDtypeStruct((M_ROWS, OUT_F), jnp.float32),
        scratch_types=[
            pltpu.VMEM((8, K), jnp.int32),
            pltpu.VMEM((8, K), jnp.int32),
            pltpu.VMEM((8, K), jnp.float32),
            pltpu.VMEM((K, HALF_F), jnp.float32),
            pltpu.VMEM((K, HALF_F), jnp.float32),
            pltpu.VMEM((K, HALF_F), jnp.float32),
            pltpu.VMEM((K, HALF_F), jnp.float32),
            pltpu.VMEM((K, HALF_F), jnp.float32),
            pltpu.VMEM((1, HALF_F), jnp.float32),
            pltpu.VMEM_SHARED((M_ROWS, HALF_F), jnp.float32),
            pltpu.VMEM_SHARED((WPAD, HALF_F), jnp.float32),
            pltpu.SemaphoreType.DMA((8,)),
            pltpu.SemaphoreType.DMA((4,)),
            pltpu.SemaphoreType.DMA((4,)),
        ],
    )
    def kfn(row_hbm, col_hbm, val_hbm, w_hbm, bias_hbm, out_hbm,
            rowb, colb, valb, g0, g1, g2, g3, zbuf, bbuf, accum, wsh,
            isem, gsem, ssem):
        c = lax.axis_index("c")
        s = lax.axis_index("s")
        gbufs = (g0, g1, g2, g3)

        wslice = WPAD // NSUB
        pltpu.sync_copy(
            w_hbm.at[pl.ds(s * wslice, wslice), pl.ds(c * HALF_F, HALF_F)],
            wsh.at[pl.ds(s * wslice, wslice)])

        pltpu.sync_copy(bias_hbm.at[pl.ds(c, 1)], bbuf)
        b0 = bbuf[0, pl.ds(0, 16)]
        b1 = bbuf[0, pl.ds(16, 16)]

        def _z(i, carry):
            zbuf[i, pl.ds(0, 16)] = b0
            zbuf[i, pl.ds(16, 16)] = b1
            return carry

        lax.fori_loop(0, K, _z, 0)

        def _fill(i, carry):
            pltpu.sync_copy(zbuf, accum.at[pl.ds(s * ROWS_PER_TILE + i * K, K)])
            return carry

        lax.fori_loop(0, ROWS_PER_TILE // K, _fill, 0)
        plsc.subcore_barrier()

        ebase = s * per_tile
        cbase = ebase

        def issue_idx(chunk, slot):
            off = chunk * K
            pltpu.async_copy(row_hbm.at[pl.ds(ebase + off, K)], rowb.at[slot],
                             isem.at[slot])
            pltpu.async_copy(col_hbm.at[pl.ds(cbase + off, K)], colb.at[slot],
                             isem.at[slot])
            pltpu.async_copy(val_hbm.at[pl.ds(ebase + off, K)], valb.at[slot],
                             isem.at[slot])

        def wait_idx(chunk, slot):
            off = chunk * K
            pltpu.make_async_copy(row_hbm.at[pl.ds(ebase + off, K)],
                                  rowb.at[slot], isem.at[slot]).wait()
            pltpu.make_async_copy(col_hbm.at[pl.ds(cbase + off, K)],
                                  colb.at[slot], isem.at[slot]).wait()
            pltpu.make_async_copy(val_hbm.at[pl.ds(ebase + off, K)],
                                  valb.at[slot], isem.at[slot]).wait()

        def issue_gather(slot8, slot4):
            pltpu.async_copy(wsh.at[colb.at[slot8]], gbufs[slot4],
                             gsem.at[slot4])

        def wait_gather(slot8, slot4):
            pltpu.make_async_copy(wsh.at[colb.at[slot8]], gbufs[slot4],
                                  gsem.at[slot4]).wait()

        def issue_scatter(slot8, slot4):
            pltpu.async_copy(gbufs[slot4], accum.at[rowb.at[slot8]],
                             ssem.at[slot4], add=True)

        def wait_scatter(slot8, slot4):
            pltpu.make_async_copy(gbufs[slot4], accum.at[rowb.at[slot8]],
                                  ssem.at[slot4]).wait()

        def multiply(slot8, slot4):
            g = gbufs[slot4]
            for grp in range(K // 16):
                vals16 = valb[slot8, pl.ds(grp * 16, 16)]
                for j in range(16):
                    e = grp * 16 + j
                    v = vals16[j]
                    g[e, pl.ds(0, 16)] = g[e, pl.ds(0, 16)] * v
                    g[e, pl.ds(16, 16)] = g[e, pl.ds(16, 16)] * v

        for j in range(6):
            issue_idx(j, j)
        wait_idx(0, 0)
        issue_gather(0, 0)
        wait_idx(1, 1)
        issue_gather(1, 1)
        issue_idx(6, 6)
        wait_idx(2, 2)
        issue_gather(2, 2)
        wait_gather(0, 0)
        multiply(0, 0)
        issue_scatter(0, 0)
        issue_idx(7, 7)
        wait_idx(3, 3)
        issue_gather(3, 3)
        wait_gather(1, 1)
        multiply(1, 1)
        issue_scatter(1, 1)

        def body(i, carry):
            cur0 = 2 + i * 8
            for u in range(8):
                cur = cur0 + u
                s8 = (2 + u) % 8
                s4 = (2 + u) % 4
                wait_scatter((s8 + 6) % 8, (s4 + 2) % 4)
                issue_idx(cur + 6, (s8 + 6) % 8)
                wait_idx(cur + 2, (s8 + 2) % 8)
                issue_gather((s8 + 2) % 8, (s4 + 2) % 4)
                wait_gather(s8, s4)
                multiply(s8, s4)
                issue_scatter(s8, s4)
            return carry

        lax.fori_loop(0, (nc - 2) // 8, body, 0)

        wait_scatter(0, 0)
        wait_scatter(1, 1)
        wait_gather(2, 2)
        wait_gather(3, 3)
        for dj in range(2, 6):
            wait_idx(nc + dj, (nc + dj) % 8)

        plsc.subcore_barrier()
        pltpu.sync_copy(
            accum.at[pl.ds(s * ROWS_PER_TILE, ROWS_PER_TILE)],
            out_hbm.at[pl.ds(s * ROWS_PER_TILE, ROWS_PER_TILE),
                       pl.ds(c * HALF_F, HALF_F)])

    return kfn


def kernel(indices, values, m, n, weight, bias):
    del m, n
    n_w = weight.shape[0]
    row = indices[0].astype(jnp.int32)
    col = indices[1].astype(jnp.int32)
    val = values.astype(jnp.float32)
    nnz = val.shape[0]

    bias2 = bias.astype(jnp.float32).reshape(2, HALF_F)

    e0 = nnz
    unit = NSUB * K
    nc = (e0 + unit - 1) // unit
    while nc % 8 != 2:
        nc += 1
    epad = unit * nc
    pad = epad - e0
    row_e = jnp.concatenate([row, jnp.zeros((pad + OVERRUN,), jnp.int32)])
    val_e = jnp.concatenate([val, jnp.zeros((pad + OVERRUN,), jnp.float32)])
    col2 = jnp.concatenate([col, jnp.zeros((pad + OVERRUN,), jnp.int32)])

    per_tile = epad // NSUB
    kfn = _spmm_kernel(n_w + 1, per_tile, nc)
    return kfn(row_e, col2, val_e, weight.astype(jnp.float32), bias2)


# speedup vs baseline: 1.1285x; 1.1285x over previous
import functools

import jax
import jax.numpy as jnp
from jax import lax
from jax.experimental import pallas as pl
from jax.experimental.pallas import tpu as pltpu
from jax.experimental.pallas import tpu_sc as plsc

M_ROWS = 16384
OUT_F = 64
HALF_F = 32
K = 128
NSUB = 16
ROWS_PER_TILE = M_ROWS // NSUB
OVERRUN = 4 * K
WPAD = 16384


@functools.lru_cache(maxsize=None)
def _spmm_kernel(n_rows_w, per_tile, nc):
    mesh = plsc.VectorSubcoreMesh(core_axis_name="c", subcore_axis_name="s")

    @functools.partial(
        pl.kernel,
        mesh=mesh,
        compiler_params=pltpu.CompilerParams(use_tc_tiling_on_sc=False),
        out_type=jax.ShapeDtypeStruct((M_ROWS, OUT_F), jnp.float32),
        scratch_types=[
            pltpu.VMEM((4, K), jnp.int32),
            pltpu.VMEM((4, K), jnp.int32),
            pltpu.VMEM((4, K), jnp.float32),
            pltpu.VMEM((K, HALF_F), jnp.float32),
            pltpu.VMEM((K, HALF_F), jnp.float32),
            pltpu.VMEM((K, HALF_F), jnp.float32),
            pltpu.VMEM((1, HALF_F), jnp.float32),
            pltpu.VMEM_SHARED((M_ROWS, HALF_F), jnp.float32),
            pltpu.VMEM_SHARED((WPAD, HALF_F), jnp.float32),
            pltpu.SemaphoreType.DMA((4,)),
            pltpu.SemaphoreType.DMA((2,)),
            pltpu.SemaphoreType.DMA((2,)),
        ],
    )
    def kfn(row_hbm, col_hbm, val_hbm, w_hbm, bias_hbm, out_hbm,
            rowb, colb, valb, g0, g1, zbuf, bbuf, accum, wsh, isem, gsem, ssem):
        c = lax.axis_index("c")
        s = lax.axis_index("s")
        gbufs = (g0, g1)

        wslice = WPAD // NSUB
        pltpu.sync_copy(
            w_hbm.at[pl.ds(s * wslice, wslice), pl.ds(c * HALF_F, HALF_F)],
            wsh.at[pl.ds(s * wslice, wslice)])

        pltpu.sync_copy(bias_hbm.at[pl.ds(c, 1)], bbuf)
        b0 = bbuf[0, pl.ds(0, 16)]
        b1 = bbuf[0, pl.ds(16, 16)]

        def _z(i, carry):
            zbuf[i, pl.ds(0, 16)] = b0
            zbuf[i, pl.ds(16, 16)] = b1
            return carry

        lax.fori_loop(0, K, _z, 0)

        def _fill(i, carry):
            pltpu.sync_copy(zbuf, accum.at[pl.ds(s * ROWS_PER_TILE + i * K, K)])
            return carry

        lax.fori_loop(0, ROWS_PER_TILE // K, _fill, 0)
        plsc.subcore_barrier()

        ebase = s * per_tile
        cbase = ebase

        def issue_idx(chunk, slot):
            off = chunk * K
            pltpu.async_copy(row_hbm.at[pl.ds(ebase + off, K)], rowb.at[slot],
                             isem.at[slot])
            pltpu.async_copy(col_hbm.at[pl.ds(cbase + off, K)], colb.at[slot],
                             isem.at[slot])
            pltpu.async_copy(val_hbm.at[pl.ds(ebase + off, K)], valb.at[slot],
                             isem.at[slot])

        def wait_idx(chunk, slot):
            off = chunk * K
            pltpu.make_async_copy(row_hbm.at[pl.ds(ebase + off, K)],
                                  rowb.at[slot], isem.at[slot]).wait()
            pltpu.make_async_copy(col_hbm.at[pl.ds(cbase + off, K)],
                                  colb.at[slot], isem.at[slot]).wait()
            pltpu.make_async_copy(val_hbm.at[pl.ds(ebase + off, K)],
                                  valb.at[slot], isem.at[slot]).wait()

        def issue_gather(slot4, slot2):
            pltpu.async_copy(wsh.at[colb.at[slot4]], gbufs[slot2],
                             gsem.at[slot2])

        def wait_gather(slot4, slot2):
            pltpu.make_async_copy(wsh.at[colb.at[slot4]], gbufs[slot2],
                                  gsem.at[slot2]).wait()

        def issue_scatter(slot4, slot2):
            pltpu.async_copy(gbufs[slot2], accum.at[rowb.at[slot4]],
                             ssem.at[slot2], add=True)

        def wait_scatter(slot4, slot2):
            pltpu.make_async_copy(gbufs[slot2], accum.at[rowb.at[slot4]],
                                  ssem.at[slot2]).wait()

        def multiply(slot4, slot2):
            g = gbufs[slot2]
            for grp in range(K // 16):
                vals16 = valb[slot4, pl.ds(grp * 16, 16)]
                for j in range(16):
                    e = grp * 16 + j
                    v = vals16[j]
                    g[e, pl.ds(0, 16)] = g[e, pl.ds(0, 16)] * v
                    g[e, pl.ds(16, 16)] = g[e, pl.ds(16, 16)] * v

        issue_idx(0, 0)
        issue_idx(1, 1)
        issue_idx(2, 2)
        wait_idx(0, 0)
        issue_gather(0, 0)
        issue_idx(3, 3)
        wait_idx(1, 1)
        issue_gather(1, 1)
        wait_gather(0, 0)
        multiply(0, 0)
        issue_scatter(0, 0)

        def body(i, carry):
            cur0 = 1 + i * 4
            for u in range(4):
                cur = cur0 + u
                s4 = (1 + u) % 4
                s2 = (1 + u) % 2
                wait_scatter((s4 + 3) % 4, (s2 + 1) % 2)
                issue_idx(cur + 3, (s4 + 3) % 4)
                wait_idx(cur + 1, (s4 + 1) % 4)
                issue_gather((s4 + 1) % 4, (s2 + 1) % 2)
                wait_gather(s4, s2)
                multiply(s4, s2)
                issue_scatter(s4, s2)
            return carry

        lax.fori_loop(0, (nc - 1) // 4, body, 0)

        wait_scatter(0, 0)
        wait_gather(1, 1)
        wait_idx(nc + 1, (nc + 1) % 4)
        wait_idx(nc + 2, (nc + 2) % 4)

        plsc.subcore_barrier()
        pltpu.sync_copy(
            accum.at[pl.ds(s * ROWS_PER_TILE, ROWS_PER_TILE)],
            out_hbm.at[pl.ds(s * ROWS_PER_TILE, ROWS_PER_TILE),
                       pl.ds(c * HALF_F, HALF_F)])

    return kfn


def kernel(indices, values, m, n, weight, bias):
    del m, n
    n_w = weight.shape[0]
    row = indices[0].astype(jnp.int32)
    col = indices[1].astype(jnp.int32)
    val = values.astype(jnp.float32)
    nnz = val.shape[0]

    bias2 = bias.astype(jnp.float32).reshape(2, HALF_F)

    e0 = nnz
    unit = NSUB * K
    nc = (e0 + unit - 1) // unit
    while nc % 4 != 1:
        nc += 1
    epad = unit * nc
    pad = epad - e0
    row_e = jnp.concatenate([row, jnp.zeros((pad + OVERRUN,), jnp.int32)])
    val_e = jnp.concatenate([val, jnp.zeros((pad + OVERRUN,), jnp.float32)])
    col2 = jnp.concatenate([col, jnp.zeros((pad + OVERRUN,), jnp.int32)])

    per_tile = epad // NSUB
    kfn = _spmm_kernel(n_w + 1, per_tile, nc)
    return kfn(row_e, col2, val_e, weight.astype(jnp.float32), bias2)
